# baseline (speedup 1.0000x reference)
import functools

import jax
import jax.numpy as jnp
from jax import lax
from jax.experimental import pallas as pl
from jax.experimental.pallas import tpu as pltpu
from jax.experimental.pallas import tpu_sc as plsc

NUM_CORES = 2
NUM_SUBCORES = 16
NUM_WORKERS = NUM_CORES * NUM_SUBCORES


def _make_lookup(batch: int, text: int, text_p: int, dim: int,
                 rows_per_chunk: int):
  assert batch % NUM_WORKERS == 0
  rows_per_w = batch // NUM_WORKERS
  assert rows_per_w % (2 * rows_per_chunk) == 0
  n_pairs = rows_per_w // (2 * rows_per_chunk)
  chunk = rows_per_chunk * text_p
  idx_per_w = rows_per_w * text_p

  mesh = plsc.VectorSubcoreMesh(core_axis_name="c", subcore_axis_name="s")

  @functools.partial(
      pl.kernel,
      mesh=mesh,
      out_type=jax.ShapeDtypeStruct((batch, text, dim), jnp.float32),
      scratch_types=[
          pltpu.VMEM((idx_per_w,), jnp.int32),
          pltpu.VMEM((chunk, dim), jnp.float32),
          pltpu.VMEM((chunk, dim), jnp.float32),
          pltpu.SemaphoreType.DMA,
          pltpu.SemaphoreType.DMA,
      ],
      compiler_params=pltpu.CompilerParams(use_tc_tiling_on_sc=True),
  )
  def lookup_kernel(table_hbm, idx_hbm, out_hbm, idx_v, buf0, buf1, sem0,
                    sem1):
    wid = lax.axis_index("s") * NUM_CORES + lax.axis_index("c")
    row_base = wid * rows_per_w
    pltpu.sync_copy(idx_hbm.at[pl.ds(row_base * text_p, idx_per_w)], idx_v)

    def gather_start(c, buf, sem):
      pltpu.async_copy(
          table_hbm.at[idx_v.at[pl.ds(c * chunk, chunk)]], buf, sem
      )

    def gather_wait(c, buf, sem):
      pltpu.make_async_copy(
          table_hbm.at[idx_v.at[pl.ds(c * chunk, chunk)]], buf, sem
      ).wait()

    def store(c, buf):
      row0 = row_base + c * rows_per_chunk
      for r in range(rows_per_chunk):
        pltpu.sync_copy(
            buf.at[pl.ds(r * text_p, text)], out_hbm.at[row0 + r]
        )

    gather_start(0, buf0, sem0)

    def body(p, carry):
      c0 = 2 * p
      gather_start(c0 + 1, buf1, sem1)
      gather_wait(c0, buf0, sem0)
      store(c0, buf0)

      @pl.when(p + 1 < n_pairs)
      def _():
        gather_start(c0 + 2, buf0, sem0)

      gather_wait(c0 + 1, buf1, sem1)
      store(c0 + 1, buf1)
      return carry

    lax.fori_loop(0, n_pairs, body, 0)

  return lookup_kernel


_lookup = _make_lookup(4096, 50, 56, 128, 8)


def kernel(input, table):
  idx = jnp.pad(input.astype(jnp.int32), ((0, 0), (0, 6))).reshape(-1)
  return _lookup(table, idx)


# speedup vs baseline: 7.5275x; 7.5275x over previous
import functools

import jax
import jax.numpy as jnp
from jax import lax
from jax.experimental import pallas as pl
from jax.experimental.pallas import tpu as pltpu
from jax.experimental.pallas import tpu_sc as plsc

NUM_CORES = 2
NUM_SUBCORES = 16
NUM_WORKERS = NUM_CORES * NUM_SUBCORES


def _make_lookup(batch: int, text: int, dim: int, rows_per_chunk: int):
  assert batch % NUM_WORKERS == 0
  rows_per_w = batch // NUM_WORKERS
  assert rows_per_w % (2 * rows_per_chunk) == 0
  n_pairs = rows_per_w // (2 * rows_per_chunk)
  chunk = rows_per_chunk * text
  idx_per_w = rows_per_w * text
  assert chunk % 8 == 0

  mesh = plsc.VectorSubcoreMesh(core_axis_name="c", subcore_axis_name="s")

  @functools.partial(
      pl.kernel,
      mesh=mesh,
      out_type=jax.ShapeDtypeStruct((batch, text, dim), jnp.float32),
      scratch_types=[
          pltpu.VMEM((idx_per_w,), jnp.int32),
          pltpu.VMEM((chunk, dim), jnp.float32),
          pltpu.VMEM((chunk, dim), jnp.float32),
          pltpu.SemaphoreType.DMA,
          pltpu.SemaphoreType.DMA,
      ],
      compiler_params=pltpu.CompilerParams(use_tc_tiling_on_sc=True),
  )
  def lookup_kernel(table_hbm, idx_hbm, out_hbm, idx_v, buf0, buf1, sem0,
                    sem1):
    wid = lax.axis_index("s") * NUM_CORES + lax.axis_index("c")
    row_base = wid * rows_per_w
    pltpu.sync_copy(idx_hbm.at[pl.ds(row_base * text, idx_per_w)], idx_v)

    def gather_start(c, buf, sem):
      pltpu.async_copy(
          table_hbm.at[idx_v.at[pl.ds(c * chunk, chunk)]], buf, sem
      )

    def gather_wait(c, buf, sem):
      pltpu.make_async_copy(
          table_hbm.at[idx_v.at[pl.ds(c * chunk, chunk)]], buf, sem
      ).wait()

    def store(c, buf):
      row0 = row_base + c * rows_per_chunk
      for r in range(rows_per_chunk):
        pltpu.sync_copy(
            buf.at[pl.ds(r * text, text)], out_hbm.at[row0 + r]
        )

    gather_start(0, buf0, sem0)

    def body(p, carry):
      c0 = 2 * p
      gather_start(c0 + 1, buf1, sem1)
      gather_wait(c0, buf0, sem0)
      store(c0, buf0)

      @pl.when(p + 1 < n_pairs)
      def _():
        gather_start(c0 + 2, buf0, sem0)

      gather_wait(c0 + 1, buf1, sem1)
      store(c0 + 1, buf1)
      return carry

    lax.fori_loop(0, n_pairs, body, 0)

  return lookup_kernel


_lookup = _make_lookup(4096, 50, 128, 8)


def kernel(input, table):
  idx = input.reshape(-1).astype(jnp.int32)
  return _lookup(table, idx)
